# Initial kernel scaffold; baseline (speedup 1.0000x reference)
import jax
import jax.numpy as jnp
from jax.experimental import pallas as pl


def kernel(x, edge_index, gcn_W, gcn_b, gru_w_ih, gru_w_hh, gru_b_ih, gru_b_hh, lin_W, lin_b):
    raise NotImplementedError("write your pallas kernel here")


# speedup vs baseline: 113.5522x; 113.5522x over previous
import functools

import jax
import jax.numpy as jnp
from jax.experimental import pallas as pl

N_ENVS = 1024
N_AG = 64
IN_DIM = 128
E_PER = 1024
G3 = 192
RNN_H = 64
OUT_DIM = 2048

BE = 8
NB = BE * N_AG


def _gcn_gru_step(cols_ref, rows_ref, x_ref, w2_ref, b2_ref, bhh_ref, out_ref):
    cols = cols_ref[...]
    rows = rows_ref[...]
    node3 = jax.lax.broadcasted_iota(jnp.int32, (BE, N_AG, E_PER), 1)
    ct = jnp.where(cols[:, None, :] == node3, 1.0, 0.0).astype(jnp.bfloat16).reshape(NB, E_PER)
    rt = jnp.where(rows[:, None, :] == node3, 1.0, 0.0).astype(jnp.bfloat16).reshape(NB, E_PER)
    a = jax.lax.dot_general(ct, rt, (((1,), (1,)), ((), ())),
                            preferred_element_type=jnp.float32)
    gi0 = jax.lax.broadcasted_iota(jnp.int32, (NB, NB), 0)
    gi1 = jax.lax.broadcasted_iota(jnp.int32, (NB, NB), 1)
    blockmask = (gi0 // N_AG) == (gi1 // N_AG)
    a = jnp.where(blockmask, a, 0.0)
    deg = jnp.sum(a, axis=1, keepdims=True) + 1.0
    dinv = jax.lax.rsqrt(deg)
    ap = a + jnp.where(gi0 == gi1, 1.0, 0.0)
    m = jnp.dot(x_ref[...], w2_ref[...], preferred_element_type=jnp.float32)
    m = m * dinv
    gi = jnp.dot(ap, m, preferred_element_type=jnp.float32) * dinv + b2_ref[...]
    bhh = bhh_ref[...]
    r = jax.nn.sigmoid(gi[:, 0:RNN_H] + bhh[:, 0:RNN_H])
    z = jax.nn.sigmoid(gi[:, RNN_H:2 * RNN_H] + bhh[:, RNN_H:2 * RNN_H])
    n = jnp.tanh(gi[:, 2 * RNN_H:] + r * bhh[:, 2 * RNN_H:])
    out_ref[...] = (1.0 - z) * n


def _head_step(a_ref, w_ref, b_ref, out_ref):
    k = pl.program_id(0)
    ab = a_ref[...].astype(jnp.bfloat16)
    wb = w_ref[...].astype(jnp.bfloat16)
    part = jax.lax.dot_general(ab, wb, (((1,), (1,)), ((), ())),
                               preferred_element_type=jnp.float32)

    @pl.when(k == 0)
    def _():
        out_ref[...] = part + b_ref[...]

    @pl.when(k != 0)
    def _():
        out_ref[...] += part


def kernel(x, edge_index, gcn_W, gcn_b, gru_w_ih, gru_w_hh, gru_b_ih, gru_b_hh, lin_W, lin_b):
    del gru_w_hh
    num_envs = x.shape[0]
    w_ih_t = gru_w_ih.T
    w2 = gcn_W @ w_ih_t
    b2 = (gcn_b @ w_ih_t + gru_b_ih)[None, :]
    bhh = gru_b_hh[None, :]
    rows = edge_index[:, 0, :]
    cols = edge_index[:, 1, :]
    x_flat = x.reshape(num_envs * N_AG, IN_DIM)

    grid1 = num_envs // BE
    h1 = pl.pallas_call(
        _gcn_gru_step,
        grid=(grid1,),
        in_specs=[
            pl.BlockSpec((BE, E_PER), lambda i: (i, 0)),
            pl.BlockSpec((BE, E_PER), lambda i: (i, 0)),
            pl.BlockSpec((NB, IN_DIM), lambda i: (i, 0)),
            pl.BlockSpec((IN_DIM, G3), lambda i: (0, 0)),
            pl.BlockSpec((1, G3), lambda i: (0, 0)),
            pl.BlockSpec((1, G3), lambda i: (0, 0)),
        ],
        out_specs=pl.BlockSpec((NB, RNN_H), lambda i: (i, 0)),
        out_shape=jax.ShapeDtypeStruct((num_envs * N_AG, RNN_H), jnp.float32),
    )(cols, rows, x_flat, w2, b2, bhh)

    rnn_out = h1.reshape(num_envs, N_AG * RNN_H)
    KB = 1024
    grid2 = (N_AG * RNN_H) // KB
    logits = pl.pallas_call(
        _head_step,
        grid=(grid2,),
        in_specs=[
            pl.BlockSpec((num_envs, KB), lambda k: (0, k)),
            pl.BlockSpec((OUT_DIM, KB), lambda k: (0, k)),
            pl.BlockSpec((1, OUT_DIM), lambda k: (0, 0)),
        ],
        out_specs=pl.BlockSpec((num_envs, OUT_DIM), lambda k: (0, 0)),
        out_shape=jax.ShapeDtypeStruct((num_envs, OUT_DIM), jnp.float32),
    )(rnn_out, lin_W, lin_b[None, :])

    next_hidden = h1.reshape(num_envs, N_AG, RNN_H)
    return (logits, next_hidden)
